# Initial kernel scaffold; baseline (speedup 1.0000x reference)
import jax
import jax.numpy as jnp
from jax.experimental import pallas as pl


def kernel(x, table, start_pos):
    raise NotImplementedError("write your pallas kernel here")


# speedup vs baseline: 1.4872x; 1.4872x over previous
import functools

import jax
import jax.numpy as jnp
from jax import lax
from jax.experimental import pallas as pl
from jax.experimental.pallas import tpu as pltpu
from jax.experimental.pallas import tpu_sc as plsc

SEQ = 8192
EMB = 1024
NUM_CORES = 2
NUM_SUBCORES = 16
NW = NUM_CORES * NUM_SUBCORES
ROWS_PER_W = SEQ // NW
CHUNK = 64
NCHUNK = ROWS_PER_W // CHUNK

_mesh = plsc.VectorSubcoreMesh(core_axis_name="c", subcore_axis_name="s")


@functools.partial(
    pl.kernel,
    mesh=_mesh,
    out_type=jax.ShapeDtypeStruct((SEQ, EMB), jnp.float32),
    scratch_types=[
        pltpu.VMEM((CHUNK,), jnp.int32),
        pltpu.VMEM((CHUNK, EMB), jnp.float32),
        pltpu.SemaphoreType.DMA,
    ],
)
def _gather_rows(table_hbm, idx_hbm, out_hbm, idx_v, rows_v, sem):
    wid = lax.axis_index("s") * NUM_CORES + lax.axis_index("c")
    base = wid * ROWS_PER_W
    for c in range(NCHUNK):
        off = base + c * CHUNK
        pltpu.sync_copy(idx_hbm.at[pl.ds(off, CHUNK)], idx_v)
        pltpu.async_copy(table_hbm.at[idx_v], rows_v, sem).wait()
        pltpu.sync_copy(rows_v, out_hbm.at[pl.ds(off, CHUNK)])


def kernel(x, table, start_pos):
    del x
    idx = jnp.asarray(start_pos, jnp.int32) + jnp.arange(SEQ, dtype=jnp.int32)
    return _gather_rows(table, idx)[None]


# speedup vs baseline: 1.4991x; 1.0080x over previous
import functools

import jax
import jax.numpy as jnp
from jax import lax
from jax.experimental import pallas as pl
from jax.experimental.pallas import tpu as pltpu
from jax.experimental.pallas import tpu_sc as plsc

SEQ = 8192
EMB = 1024
NUM_CORES = 2
NUM_SUBCORES = 16
NW = NUM_CORES * NUM_SUBCORES
ROWS_PER_W = SEQ // NW
CHUNK = 32
NCHUNK = ROWS_PER_W // CHUNK

_mesh = plsc.VectorSubcoreMesh(core_axis_name="c", subcore_axis_name="s")


@functools.partial(
    pl.kernel,
    mesh=_mesh,
    out_type=jax.ShapeDtypeStruct((SEQ, EMB), jnp.float32),
    scratch_types=[
        pltpu.VMEM((ROWS_PER_W,), jnp.int32),
        pltpu.VMEM((CHUNK, EMB), jnp.float32),
        pltpu.VMEM((CHUNK, EMB), jnp.float32),
        pltpu.SemaphoreType.DMA,
        pltpu.SemaphoreType.DMA,
    ],
)
def _gather_rows(table_hbm, idx_hbm, out_hbm, idx_v, buf0, buf1, g_sem, w_sem):
    wid = lax.axis_index("s") * NUM_CORES + lax.axis_index("c")
    base = wid * ROWS_PER_W
    pltpu.sync_copy(idx_hbm.at[pl.ds(base, ROWS_PER_W)], idx_v)
    bufs = (buf0, buf1)

    def start_gather(c):
        return pltpu.async_copy(
            table_hbm.at[idx_v.at[pl.ds(c * CHUNK, CHUNK)]], bufs[c % 2], g_sem)

    gathers = [start_gather(0)]
    writes = [None] * NCHUNK
    for c in range(NCHUNK):
        gathers[c].wait()
        if c + 1 < NCHUNK:
            if c >= 1:
                writes[c - 1].wait()
            gathers.append(start_gather(c + 1))
        writes[c] = pltpu.async_copy(
            bufs[c % 2], out_hbm.at[pl.ds(base + c * CHUNK, CHUNK)], w_sem)
    writes[NCHUNK - 2].wait()
    writes[NCHUNK - 1].wait()


def kernel(x, table, start_pos):
    del x
    idx = jnp.asarray(start_pos, jnp.int32) + jnp.arange(SEQ, dtype=jnp.int32)
    return _gather_rows(table, idx)[None]


# speedup vs baseline: 1.5180x; 1.0126x over previous
import functools

import jax
import jax.numpy as jnp
from jax import lax
from jax.experimental import pallas as pl
from jax.experimental.pallas import tpu as pltpu
from jax.experimental.pallas import tpu_sc as plsc

SEQ = 8192
EMB = 1024
NUM_CORES = 2
NUM_SUBCORES = 16
LANES = 16
NW = NUM_CORES * NUM_SUBCORES
ROWS_PER_W = SEQ // NW
CHUNK = 32
NCHUNK = ROWS_PER_W // CHUNK

_mesh = plsc.VectorSubcoreMesh(core_axis_name="c", subcore_axis_name="s")


@functools.partial(
    pl.kernel,
    mesh=_mesh,
    out_type=jax.ShapeDtypeStruct((SEQ, EMB), jnp.float32),
    scratch_types=[
        pltpu.VMEM((LANES,), jnp.int32),
        pltpu.VMEM((ROWS_PER_W,), jnp.int32),
        pltpu.VMEM((CHUNK, EMB), jnp.float32),
        pltpu.VMEM((CHUNK, EMB), jnp.float32),
        pltpu.SemaphoreType.DMA,
        pltpu.SemaphoreType.DMA,
    ],
)
def _gather_rows(table_hbm, sp_hbm, out_hbm, sp_v, idx_v, buf0, buf1,
                 g_sem, w_sem):
    wid = lax.axis_index("s") * NUM_CORES + lax.axis_index("c")
    base = wid * ROWS_PER_W
    pltpu.sync_copy(sp_hbm, sp_v)
    lane = lax.iota(jnp.int32, LANES)
    start_vec = sp_v[...]
    for k in range(ROWS_PER_W // LANES):
        idx_v[pl.ds(k * LANES, LANES)] = start_vec + (base + k * LANES) + lane
    bufs = (buf0, buf1)

    def start_gather(c):
        return pltpu.async_copy(
            table_hbm.at[idx_v.at[pl.ds(c * CHUNK, CHUNK)]], bufs[c % 2], g_sem)

    gathers = [start_gather(0)]
    writes = [None] * NCHUNK
    for c in range(NCHUNK):
        gathers[c].wait()
        if c + 1 < NCHUNK:
            if c >= 1:
                writes[c - 1].wait()
            gathers.append(start_gather(c + 1))
        writes[c] = pltpu.async_copy(
            bufs[c % 2], out_hbm.at[pl.ds(base + c * CHUNK, CHUNK)], w_sem)
    writes[NCHUNK - 2].wait()
    writes[NCHUNK - 1].wait()


def kernel(x, table, start_pos):
    del x
    sp = jnp.full((LANES,), start_pos, jnp.int32)
    return _gather_rows(table, sp)[None]
